# baseline (device time: 21262 ns/iter reference)
import jax
import jax.numpy as jnp
from jax import lax
from jax.experimental import pallas as pl
from jax.experimental.pallas import tpu as pltpu

N_DEV = 4
E_PER_DEV = 4
N_EXPERTS = 16
E_HALF = E_PER_DEV // 2

W_CLIP = 0.1
Q_INV = 127.0 / W_CLIP
Q_DEQ = W_CLIP / 127.0


def kernel(x, router_W, route_idx, expert_W, shared_W):
    n_tok, d_model = x.shape
    d_ff = shared_W.shape[1]

    def body(x_ref, router_ref, idx_ref, ew_ref, sw_ref, out_ref,
             ewq_ref, commA_ref, commB_ref,
             sendA, recvA, sendB, recvB):
        my_pos = lax.axis_index("i")
        left = (my_pos - 1) % N_DEV
        right = (my_pos + 1) % N_DEV

        barrier_sem = pltpu.get_barrier_semaphore()
        for nbr in [left, right]:
            pl.semaphore_signal(
                barrier_sem, inc=1,
                device_id=(nbr,), device_id_type=pl.DeviceIdType.MESH,
            )

        for e in range(E_PER_DEV):
            ewq_ref[e, :, :] = jnp.clip(
                jnp.round(ew_ref[e, :, :] * Q_INV),
                -127.0, 127.0).astype(jnp.int8)

        pl.semaphore_wait(barrier_sem, 2)

        def make_hop(h, j, comm_ref, send_sems, recv_sems, first_src, dst_dev):
            return pltpu.make_async_remote_copy(
                src_ref=first_src if h == 0 else comm_ref.at[h - 1, j],
                dst_ref=comm_ref.at[h, j],
                send_sem=send_sems.at[h, j],
                recv_sem=recv_sems.at[h, j],
                device_id=(dst_dev,),
                device_id_type=pl.DeviceIdType.MESH,
            )

        rdmasA = [make_hop(0, j, commA_ref, sendA, recvA,
                           ewq_ref.at[j], right)
                  for j in range(E_HALF)]
        rdmasB = [make_hop(0, j, commB_ref, sendB, recvB,
                           ewq_ref.at[E_HALF + j], left)
                  for j in range(E_HALF)]
        for r in rdmasA + rdmasB:
            r.start()

        xv = x_ref[:, :]
        scores = jnp.dot(xv, router_ref[:, :],
                         preferred_element_type=jnp.float32)
        s_max = jnp.max(scores, axis=-1, keepdims=True)
        probs = jnp.exp(scores - s_max)
        probs = probs / jnp.sum(probs, axis=-1, keepdims=True)

        idx = idx_ref[:, :]
        lane = lax.broadcasted_iota(jnp.int32, (n_tok, N_EXPERTS), 1)
        onehot = (lane == idx).astype(jnp.float32)
        p_sel = jnp.sum(probs * onehot, axis=-1, keepdims=True)

        out_ref[:, :] = jnp.dot(xv, sw_ref[:, :],
                                preferred_element_type=jnp.float32)

        def scaled_x(glob_e, scale):
            coef = (p_sel * scale) * (idx == glob_e).astype(jnp.float32)
            return (xv * coef).astype(jnp.bfloat16)

        def add_experts_fused(w_stack_bf, glob_es, scale):
            xs = jnp.concatenate(
                [scaled_x(g, scale) for g in glob_es], axis=1)
            out_ref[:, :] += jnp.dot(
                xs, w_stack_bf,
                preferred_element_type=jnp.float32,
            )

        add_experts_fused(
            jnp.reshape(ewq_ref[:, :, :].astype(jnp.bfloat16),
                        (E_PER_DEV * d_model, d_ff)),
            [my_pos * E_PER_DEV + k for k in range(E_PER_DEV)],
            Q_DEQ,
        )

        for h in range(N_DEV - 1):
            originA = (my_pos - h - 1) % N_DEV
            originB = (my_pos + h + 1) % N_DEV
            for j in range(E_HALF):
                rdmasA[h * E_HALF + j].wait_recv()
                if h + 1 < N_DEV - 1:
                    nxt = make_hop(h + 1, j, commA_ref, sendA, recvA,
                                   None, right)
                    nxt.start()
                    rdmasA.append(nxt)
                rdmasB[h * E_HALF + j].wait_recv()
                if h + 1 < N_DEV - 1:
                    nxt = make_hop(h + 1, j, commB_ref, sendB, recvB,
                                   None, left)
                    nxt.start()
                    rdmasB.append(nxt)
            add_experts_fused(
                jnp.reshape(commA_ref[h, :, :, :].astype(jnp.bfloat16),
                            (E_HALF * d_model, d_ff)),
                [originA * E_PER_DEV + j for j in range(E_HALF)],
                Q_DEQ,
            )
            add_experts_fused(
                jnp.reshape(commB_ref[h, :, :, :].astype(jnp.bfloat16),
                            (E_HALF * d_model, d_ff)),
                [originB * E_PER_DEV + E_HALF + j for j in range(E_HALF)],
                Q_DEQ,
            )

        for rdma in rdmasA + rdmasB:
            rdma.wait_send()

    return pl.pallas_call(
        body,
        out_shape=jax.ShapeDtypeStruct((n_tok, d_ff), jnp.float32),
        in_specs=[
            pl.BlockSpec(memory_space=pltpu.VMEM),
            pl.BlockSpec(memory_space=pltpu.VMEM),
            pl.BlockSpec(memory_space=pltpu.VMEM),
            pl.BlockSpec(memory_space=pltpu.VMEM),
            pl.BlockSpec(memory_space=pltpu.VMEM),
        ],
        out_specs=pl.BlockSpec(memory_space=pltpu.VMEM),
        scratch_shapes=[
            pltpu.VMEM((E_PER_DEV, d_model, d_ff), jnp.int8),
            pltpu.VMEM((N_DEV - 1, E_HALF, d_model, d_ff), jnp.int8),
            pltpu.VMEM((N_DEV - 1, E_HALF, d_model, d_ff), jnp.int8),
            pltpu.SemaphoreType.DMA((N_DEV - 1, E_HALF)),
            pltpu.SemaphoreType.DMA((N_DEV - 1, E_HALF)),
            pltpu.SemaphoreType.DMA((N_DEV - 1, E_HALF)),
            pltpu.SemaphoreType.DMA((N_DEV - 1, E_HALF)),
        ],
        compiler_params=pltpu.CompilerParams(collective_id=0),
    )(x, router_W, route_idx, expert_W, shared_W)


# device time: 20879 ns/iter; 1.0183x vs baseline; 1.0183x over previous
import jax
import jax.numpy as jnp
from jax import lax
from jax.experimental import pallas as pl
from jax.experimental.pallas import tpu as pltpu

N_DEV = 4
E_PER_DEV = 4
N_EXPERTS = 16
E_HALF = E_PER_DEV // 2

W_CLIP = 0.1
Q_INV = 127.0 / W_CLIP
Q_DEQ = W_CLIP / 127.0


def kernel(x, router_W, route_idx, expert_W, shared_W):
    n_tok, d_model = x.shape
    d_ff = shared_W.shape[1]

    def body(x_ref, router_ref, idx_ref, ew_ref, sw_ref, out_ref,
             ewq_ref, commA_ref, commB_ref,
             sendA, recvA, sendB, recvB):
        my_pos = lax.axis_index("i")
        left = (my_pos - 1) % N_DEV
        right = (my_pos + 1) % N_DEV

        barrier_sem = pltpu.get_barrier_semaphore()
        for nbr in [left, right]:
            pl.semaphore_signal(
                barrier_sem, inc=1,
                device_id=(nbr,), device_id_type=pl.DeviceIdType.MESH,
            )

        for e in range(E_PER_DEV):
            ewq_ref[e, :, :] = jnp.clip(
                jnp.round(ew_ref[e, :, :] * Q_INV),
                -127.0, 127.0).astype(jnp.int8)

        pl.semaphore_wait(barrier_sem, 2)

        def make_hop(h, j, comm_ref, send_sems, recv_sems, first_src, dst_dev):
            return pltpu.make_async_remote_copy(
                src_ref=first_src if h == 0 else comm_ref.at[h - 1, j],
                dst_ref=comm_ref.at[h, j],
                send_sem=send_sems.at[h, j],
                recv_sem=recv_sems.at[h, j],
                device_id=(dst_dev,),
                device_id_type=pl.DeviceIdType.MESH,
            )

        rdmasA = [make_hop(0, j, commA_ref, sendA, recvA,
                           ewq_ref.at[j], right)
                  for j in range(E_HALF)]
        rdmasB = [make_hop(0, j, commB_ref, sendB, recvB,
                           ewq_ref.at[E_HALF + j], left)
                  for j in range(E_HALF)]
        for r in rdmasA + rdmasB:
            r.start()

        xv = x_ref[:, :]
        scores = jnp.dot(xv, router_ref[:, :],
                         preferred_element_type=jnp.float32)
        s_max = jnp.max(scores, axis=-1, keepdims=True)
        probs = jnp.exp(scores - s_max)
        probs = probs / jnp.sum(probs, axis=-1, keepdims=True)

        idx = idx_ref[:, :]
        lane = lax.broadcasted_iota(jnp.int32, (n_tok, N_EXPERTS), 1)
        onehot = (lane == idx).astype(jnp.float32)
        p_sel = jnp.sum(probs * onehot, axis=-1, keepdims=True)

        out_ref[:, :] = jnp.dot(xv, sw_ref[:, :],
                                preferred_element_type=jnp.float32)

        def add_expert(w_bf, glob_e, scale):
            coef = (p_sel * scale) * (idx == glob_e).astype(jnp.float32)
            xs = (xv * coef).astype(jnp.bfloat16)
            out_ref[:, :] += jnp.dot(
                xs, w_bf,
                preferred_element_type=jnp.float32,
            )

        for h in range(N_DEV - 1):
            originA = (my_pos - h - 1) % N_DEV
            originB = (my_pos + h + 1) % N_DEV
            for j in range(E_HALF):
                rdmasA[h * E_HALF + j].wait_recv()
                if h + 1 < N_DEV - 1:
                    nxt = make_hop(h + 1, j, commA_ref, sendA, recvA,
                                   None, right)
                    nxt.start()
                    rdmasA.append(nxt)
                rdmasB[h * E_HALF + j].wait_recv()
                if h + 1 < N_DEV - 1:
                    nxt = make_hop(h + 1, j, commB_ref, sendB, recvB,
                                   None, left)
                    nxt.start()
                    rdmasB.append(nxt)
                if h == 0:
                    for k in (2 * j, 2 * j + 1):
                        glob_e = my_pos * E_PER_DEV + k
                        coef = p_sel * (idx == glob_e).astype(jnp.float32)
                        out_ref[:, :] += jnp.dot(
                            xv * coef, ew_ref[k, :, :],
                            preferred_element_type=jnp.float32,
                        )
                add_expert(commA_ref[h, j, :, :].astype(jnp.bfloat16),
                           originA * E_PER_DEV + j, Q_DEQ)
                add_expert(commB_ref[h, j, :, :].astype(jnp.bfloat16),
                           originB * E_PER_DEV + E_HALF + j, Q_DEQ)

        for rdma in rdmasA + rdmasB:
            rdma.wait_send()

    return pl.pallas_call(
        body,
        out_shape=jax.ShapeDtypeStruct((n_tok, d_ff), jnp.float32),
        in_specs=[
            pl.BlockSpec(memory_space=pltpu.VMEM),
            pl.BlockSpec(memory_space=pltpu.VMEM),
            pl.BlockSpec(memory_space=pltpu.VMEM),
            pl.BlockSpec(memory_space=pltpu.VMEM),
            pl.BlockSpec(memory_space=pltpu.VMEM),
        ],
        out_specs=pl.BlockSpec(memory_space=pltpu.VMEM),
        scratch_shapes=[
            pltpu.VMEM((E_PER_DEV, d_model, d_ff), jnp.int8),
            pltpu.VMEM((N_DEV - 1, E_HALF, d_model, d_ff), jnp.int8),
            pltpu.VMEM((N_DEV - 1, E_HALF, d_model, d_ff), jnp.int8),
            pltpu.SemaphoreType.DMA((N_DEV - 1, E_HALF)),
            pltpu.SemaphoreType.DMA((N_DEV - 1, E_HALF)),
            pltpu.SemaphoreType.DMA((N_DEV - 1, E_HALF)),
            pltpu.SemaphoreType.DMA((N_DEV - 1, E_HALF)),
        ],
        compiler_params=pltpu.CompilerParams(collective_id=0),
    )(x, router_W, route_idx, expert_W, shared_W)
